# baseline (speedup 1.0000x reference)
import functools

import jax
import jax.numpy as jnp
from jax import lax
from jax.experimental import pallas as pl
from jax.experimental.pallas import tpu as pltpu
from jax.experimental.pallas import tpu_sc as plsc

N = 10000
E = 320000
D = 128
OUT = 128
G = 64
STEPS = 3

NC = 2
NS = 16
NW = NC * NS
K = 128
NCHUNK = 80
EPW = NCHUNK * K
EPAD = NW * EPW
NRCZ = -(-N // K) + 1
AGGR = NRCZ * K
RB = 80
NRC = N // RB
RC_PER_TILE = -(-NRC // NS)

ROWS_B = 1000
NBLK = N // ROWS_B


_sc_mesh = plsc.VectorSubcoreMesh(core_axis_name="c", subcore_axis_name="s")


@functools.partial(
    pl.kernel,
    out_type=jax.ShapeDtypeStruct((NC * N, D), jnp.float32),
    mesh=_sc_mesh,
    scratch_types=[
        pltpu.VMEM((NCHUNK, K), jnp.int32),
        [pltpu.VMEM((K,), jnp.int32) for _ in range(4)],
        pltpu.VMEM((K, D), jnp.float32),
        pltpu.VMEM((K, D), jnp.float32),
        pltpu.VMEM_SHARED((AGGR, D), jnp.float32),
        [pltpu.SemaphoreType.DMA for _ in range(4)],
        pltpu.SemaphoreType.DMA,
        pltpu.SemaphoreType.DMA,
        pltpu.SemaphoreType.DMA,
    ],
)
def _sc_gather_segsum(msg_hbm, src_hbm, dst_hbm, out_hbm,
                      dst_all, ibufs, rows0, rows1, agg_sh,
                      semi, semg0, semg1, semz):
    c = lax.axis_index("c")
    s = lax.axis_index("s")
    wid = s * NC + c
    base_w = pl.multiple_of(wid * EPW, EPW)
    rows = (rows0, rows1)
    semg = (semg0, semg1)

    zeros16 = jnp.zeros((16,), jnp.float32)

    def _zero_row(r, carry):
        for j in range(D // 16):
            rows0[r, pl.ds(j * 16, 16)] = zeros16
        return carry

    lax.fori_loop(0, K, _zero_row, 0)

    def _zero_chunk_sync(t, carry):
        j = s + t * NS

        @pl.when(j < NRCZ)
        def _():
            pltpu.sync_copy(rows0, agg_sh.at[pl.ds(pl.multiple_of(j * K, K), K)])

        return carry

    lax.fori_loop(0, -(-NRCZ // NS), _zero_chunk_sync, 0)

    pltpu.sync_copy(dst_hbm.at[wid], dst_all)
    plsc.subcore_barrier()

    def _load_idx(i, j):
        pltpu.async_copy(src_hbm.at[pl.ds(base_w + pl.multiple_of(i * K, K), K)],
                         ibufs[j], semi[j])

    def _issue_gather(i, b, j):
        pltpu.make_async_copy(src_hbm.at[pl.ds(0, K)], ibufs[j], semi[j]).wait()
        pltpu.async_copy(msg_hbm.at[ibufs[j]], rows[b], semg[b])

    def _drain_scatter(i, b):
        pltpu.make_async_copy(msg_hbm.at[pl.ds(0, K)], rows[b], semg[b]).wait()
        pltpu.sync_copy(rows[b], agg_sh.at[dst_all.at[i]], add=True)

    for j in range(4):
        _load_idx(j, j)
    _issue_gather(0, 0, 0)
    _issue_gather(1, 1, 1)

    def _edge_quad(t, carry):
        for b in range(4):
            i = 4 * t + b
            _drain_scatter(i, b % 2)

            @pl.when(i + 4 < NCHUNK)
            def _():
                _load_idx(i + 4, b)

            @pl.when(i + 2 < NCHUNK)
            def _():
                _issue_gather(i + 2, b % 2, (b + 2) % 4)

        return carry

    lax.fori_loop(0, NCHUNK // 4, _edge_quad, 0)
    plsc.subcore_barrier()

    def _write_chunk(t, carry):
        j = s + t * NS

        @pl.when(j < NRC)
        def _():
            base = pl.multiple_of(j * RB, RB)
            pltpu.async_copy(agg_sh.at[pl.ds(base, RB)],
                             out_hbm.at[pl.ds(pl.multiple_of(c * N, RB) + base, RB)],
                             semz)

        return carry

    def _write_drain(t, carry):
        j = s + t * NS

        @pl.when(j < NRC)
        def _():
            base = pl.multiple_of(j * RB, RB)
            pltpu.make_async_copy(agg_sh.at[pl.ds(base, RB)],
                                  out_hbm.at[pl.ds(pl.multiple_of(c * N, RB) + base, RB)],
                                  semz).wait()

        return carry

    lax.fori_loop(0, RC_PER_TILE, _write_chunk, 0)
    lax.fori_loop(0, RC_PER_TILE, _write_drain, 0)


def _tc_a_body(x_ref, w_ref, im_ref, msg_ref):
    im = lax.dot_general(x_ref[...], w_ref[...], (((1,), (1,)), ((), ())),
                         preferred_element_type=jnp.float32)
    im_ref[...] = im
    msg_ref[...] = jnp.maximum(im, 0.0)


_tc_a = pl.pallas_call(
    _tc_a_body,
    grid=(NBLK,),
    in_specs=[
        pl.BlockSpec((ROWS_B, D), lambda i: (i, 0)),
        pl.BlockSpec((OUT, D), lambda i: (0, 0)),
    ],
    out_specs=[
        pl.BlockSpec((ROWS_B, OUT), lambda i: (i, 0)),
        pl.BlockSpec((ROWS_B, OUT), lambda i: (i, 0)),
    ],
    out_shape=[
        jax.ShapeDtypeStruct((N, OUT), jnp.float32),
        jax.ShapeDtypeStruct((N, OUT), jnp.float32),
    ],
)


def _tc_b_body(im_ref, a_ref, b_ref, w_ref, msg_ref):
    agg = a_ref[...] + b_ref[...]
    y = lax.dot_general(agg, w_ref[...], (((1,), (1,)), ((), ())),
                        preferred_element_type=jnp.float32)
    msg_ref[...] = jnp.maximum(im_ref[...] + y, 0.0)


_tc_b = pl.pallas_call(
    _tc_b_body,
    grid=(NBLK,),
    in_specs=[
        pl.BlockSpec((ROWS_B, OUT), lambda i: (i, 0)),
        pl.BlockSpec((ROWS_B, OUT), lambda i: (i, 0)),
        pl.BlockSpec((ROWS_B, OUT), lambda i: (i, 0)),
        pl.BlockSpec((OUT, OUT), lambda i: (0, 0)),
    ],
    out_specs=pl.BlockSpec((ROWS_B, OUT), lambda i: (i, 0)),
    out_shape=jax.ShapeDtypeStruct((N, OUT), jnp.float32),
)


def _tc_b_last_body(im_ref, a_ref, b_ref, w_ref, gid_ref, out_ref):
    i = pl.program_id(0)
    agg = a_ref[...] + b_ref[...]
    y = lax.dot_general(agg, w_ref[...], (((1,), (1,)), ((), ())),
                        preferred_element_type=jnp.float32)
    msg = jnp.maximum(im_ref[...] + y, 0.0)
    gid = gid_ref[0, 0, :]
    graphs = lax.broadcasted_iota(jnp.int32, (ROWS_B, G), 1)
    onehot = jnp.where(gid[:, None] == graphs, 1.0, 0.0).astype(jnp.float32)
    contrib = lax.dot_general(onehot, msg, (((0,), (0,)), ((), ())),
                              preferred_element_type=jnp.float32)

    @pl.when(i == 0)
    def _():
        out_ref[...] = jnp.zeros_like(out_ref)

    out_ref[...] += contrib


_tc_b_last = pl.pallas_call(
    _tc_b_last_body,
    grid=(NBLK,),
    in_specs=[
        pl.BlockSpec((ROWS_B, OUT), lambda i: (i, 0)),
        pl.BlockSpec((ROWS_B, OUT), lambda i: (i, 0)),
        pl.BlockSpec((ROWS_B, OUT), lambda i: (i, 0)),
        pl.BlockSpec((OUT, OUT), lambda i: (0, 0)),
        pl.BlockSpec((1, 1, ROWS_B), lambda i: (i, 0, 0)),
    ],
    out_specs=pl.BlockSpec((G, OUT), lambda i: (0, 0)),
    out_shape=jax.ShapeDtypeStruct((G, OUT), jnp.float32),
)


def kernel(node_feat, edge_index, graph_ids, W_n2l, W_rec):
    pad = jnp.arange(EPAD - E, dtype=jnp.int32)
    src = jnp.concatenate([edge_index[0], pad % N])
    dst = jnp.concatenate(
        [edge_index[1], N + pad % (AGGR - N)]).reshape(NW, NCHUNK, K)
    gid3 = graph_ids.reshape(NBLK, 1, ROWS_B)

    im, msg = _tc_a(node_feat, W_n2l)
    for step in range(STEPS):
        parts = _sc_gather_segsum(msg, src, dst)
        agg_a = parts[:N]
        agg_b = parts[N:]
        if step < STEPS - 1:
            msg = _tc_b(im, agg_a, agg_b, W_rec)
        else:
            out = _tc_b_last(im, agg_a, agg_b, W_rec, gid3)
    return out


# speedup vs baseline: 1.0087x; 1.0087x over previous
import functools

import jax
import jax.numpy as jnp
from jax import lax
from jax.experimental import pallas as pl
from jax.experimental.pallas import tpu as pltpu
from jax.experimental.pallas import tpu_sc as plsc

N = 10000
E = 320000
D = 128
OUT = 128
G = 64
STEPS = 3

NC = 2
NS = 16
NW = NC * NS
K = 128
NCHUNK = 80
EPW = NCHUNK * K
EPAD = NW * EPW
NRCZ = -(-N // K) + 1
AGGR = NRCZ * K
RB = 80
NRC = N // RB
RC_PER_TILE = -(-NRC // NS)

ROWS_B = 1000
NBLK = N // ROWS_B


_sc_mesh = plsc.VectorSubcoreMesh(core_axis_name="c", subcore_axis_name="s")


@functools.partial(
    pl.kernel,
    out_type=jax.ShapeDtypeStruct((NC * N, D), jnp.float32),
    mesh=_sc_mesh,
    scratch_types=[
        pltpu.VMEM((EPW,), jnp.int32),
        pltpu.VMEM((K,), jnp.int32),
        pltpu.VMEM((K,), jnp.int32),
        pltpu.VMEM((K, D), jnp.float32),
        pltpu.VMEM((K, D), jnp.float32),
        pltpu.VMEM_SHARED((AGGR, D), jnp.float32),
        pltpu.SemaphoreType.DMA,
        pltpu.SemaphoreType.DMA,
        pltpu.SemaphoreType.DMA,
        pltpu.SemaphoreType.DMA,
    ],
)
def _sc_gather_segsum(msg_hbm, src_hbm, dst_hbm, out_hbm,
                      src_all, dst_c0, dst_c1, rows0, rows1, agg_sh,
                      semd0, semd1, semg0, semg1):
    c = lax.axis_index("c")
    s = lax.axis_index("s")
    wid = s * NC + c
    base_w = pl.multiple_of(wid * EPW, EPW)

    zeros16 = jnp.zeros((16,), jnp.float32)

    def _zero_row(r, carry):
        for j in range(D // 16):
            rows0[r, pl.ds(j * 16, 16)] = zeros16
        return carry

    lax.fori_loop(0, K, _zero_row, 0)

    def _zero_chunk(t, carry):
        j = s + t * NS
        pltpu.sync_copy(rows0, agg_sh.at[pl.ds(pl.multiple_of(j * K, K), K)])
        return carry

    lax.fori_loop(0, NRCZ // NS, _zero_chunk, 0)

    pltpu.sync_copy(src_hbm.at[pl.ds(base_w, EPW)], src_all)
    plsc.subcore_barrier()

    def _issue(i, dst_c, rows, semd, semg):
        pltpu.async_copy(dst_hbm.at[pl.ds(base_w + pl.multiple_of(i * K, K), K)],
                         dst_c, semd)
        idx = src_all.at[pl.ds(pl.multiple_of(i * K, K), K)]
        pltpu.async_copy(msg_hbm.at[idx], rows, semg)

    def _drain_scatter(dst_c, rows, semd, semg):
        pltpu.make_async_copy(dst_hbm.at[pl.ds(0, K)], dst_c, semd).wait()
        pltpu.make_async_copy(msg_hbm.at[pl.ds(0, K)], rows, semg).wait()
        pltpu.sync_copy(rows, agg_sh.at[dst_c], add=True)

    _issue(0, dst_c0, rows0, semd0, semg0)
    _issue(1, dst_c1, rows1, semd1, semg1)

    def _edge_pair(t, carry):
        _drain_scatter(dst_c0, rows0, semd0, semg0)
        _issue(2 * t + 2, dst_c0, rows0, semd0, semg0)
        _drain_scatter(dst_c1, rows1, semd1, semg1)
        _issue(2 * t + 3, dst_c1, rows1, semd1, semg1)
        return carry

    lax.fori_loop(0, NCHUNK // 2 - 1, _edge_pair, 0)
    _drain_scatter(dst_c0, rows0, semd0, semg0)
    _drain_scatter(dst_c1, rows1, semd1, semg1)
    plsc.subcore_barrier()

    def _write_chunk(t, carry):
        j = s + t * NS

        @pl.when(j < NRC)
        def _():
            base = pl.multiple_of(j * RB, RB)
            pltpu.sync_copy(agg_sh.at[pl.ds(base, RB)],
                            out_hbm.at[pl.ds(pl.multiple_of(c * N, RB) + base, RB)])

        return carry

    lax.fori_loop(0, RC_PER_TILE, _write_chunk, 0)


def _tc_a_body(x_ref, w_ref, im_ref, msg_ref):
    im = lax.dot_general(x_ref[...], w_ref[...], (((1,), (1,)), ((), ())),
                         preferred_element_type=jnp.float32)
    im_ref[...] = im
    msg_ref[...] = jnp.maximum(im, 0.0)


_tc_a = pl.pallas_call(
    _tc_a_body,
    grid=(NBLK,),
    in_specs=[
        pl.BlockSpec((ROWS_B, D), lambda i: (i, 0)),
        pl.BlockSpec((OUT, D), lambda i: (0, 0)),
    ],
    out_specs=[
        pl.BlockSpec((ROWS_B, OUT), lambda i: (i, 0)),
        pl.BlockSpec((ROWS_B, OUT), lambda i: (i, 0)),
    ],
    out_shape=[
        jax.ShapeDtypeStruct((N, OUT), jnp.float32),
        jax.ShapeDtypeStruct((N, OUT), jnp.float32),
    ],
)


def _tc_b_body(im_ref, a_ref, b_ref, w_ref, msg_ref):
    agg = a_ref[...] + b_ref[...]
    y = lax.dot_general(agg, w_ref[...], (((1,), (1,)), ((), ())),
                        preferred_element_type=jnp.float32)
    msg_ref[...] = jnp.maximum(im_ref[...] + y, 0.0)


_tc_b = pl.pallas_call(
    _tc_b_body,
    grid=(NBLK,),
    in_specs=[
        pl.BlockSpec((ROWS_B, OUT), lambda i: (i, 0)),
        pl.BlockSpec((ROWS_B, OUT), lambda i: (i, 0)),
        pl.BlockSpec((ROWS_B, OUT), lambda i: (i, 0)),
        pl.BlockSpec((OUT, OUT), lambda i: (0, 0)),
    ],
    out_specs=pl.BlockSpec((ROWS_B, OUT), lambda i: (i, 0)),
    out_shape=jax.ShapeDtypeStruct((N, OUT), jnp.float32),
)


def _tc_b_last_body(im_ref, a_ref, b_ref, w_ref, gid_ref, out_ref):
    i = pl.program_id(0)
    agg = a_ref[...] + b_ref[...]
    y = lax.dot_general(agg, w_ref[...], (((1,), (1,)), ((), ())),
                        preferred_element_type=jnp.float32)
    msg = jnp.maximum(im_ref[...] + y, 0.0)
    gid = gid_ref[0, 0, :]
    graphs = lax.broadcasted_iota(jnp.int32, (ROWS_B, G), 1)
    onehot = jnp.where(gid[:, None] == graphs, 1.0, 0.0).astype(jnp.float32)
    contrib = lax.dot_general(onehot, msg, (((0,), (0,)), ((), ())),
                              preferred_element_type=jnp.float32)

    @pl.when(i == 0)
    def _():
        out_ref[...] = jnp.zeros_like(out_ref)

    out_ref[...] += contrib


_tc_b_last = pl.pallas_call(
    _tc_b_last_body,
    grid=(NBLK,),
    in_specs=[
        pl.BlockSpec((ROWS_B, OUT), lambda i: (i, 0)),
        pl.BlockSpec((ROWS_B, OUT), lambda i: (i, 0)),
        pl.BlockSpec((ROWS_B, OUT), lambda i: (i, 0)),
        pl.BlockSpec((OUT, OUT), lambda i: (0, 0)),
        pl.BlockSpec((1, 1, ROWS_B), lambda i: (i, 0, 0)),
    ],
    out_specs=pl.BlockSpec((G, OUT), lambda i: (0, 0)),
    out_shape=jax.ShapeDtypeStruct((G, OUT), jnp.float32),
)


def kernel(node_feat, edge_index, graph_ids, W_n2l, W_rec):
    pad = jnp.arange(EPAD - E, dtype=jnp.int32)
    src = jnp.concatenate([edge_index[0], pad % N])
    dst = jnp.concatenate([edge_index[1], N + pad % (AGGR - N)])
    gid3 = graph_ids.reshape(NBLK, 1, ROWS_B)

    im, msg = _tc_a(node_feat, W_n2l)
    for step in range(STEPS):
        parts = _sc_gather_segsum(msg, src, dst)
        agg_a = parts[:N]
        agg_b = parts[N:]
        if step < STEPS - 1:
            msg = _tc_b(im, agg_a, agg_b, W_rec)
        else:
            out = _tc_b_last(im, agg_a, agg_b, W_rec, gid3)
    return out


# speedup vs baseline: 1.0142x; 1.0054x over previous
import functools

import jax
import jax.numpy as jnp
from jax import lax
from jax.experimental import pallas as pl
from jax.experimental.pallas import tpu as pltpu
from jax.experimental.pallas import tpu_sc as plsc

N = 10000
E = 320000
D = 128
OUT = 128
G = 64
STEPS = 3

NC = 2
NS = 16
NW = NC * NS
K = 128
NCHUNK = 80
EPW = NCHUNK * K
EPAD = NW * EPW
NRCZ = -(-N // K) + 1
AGGR = NRCZ * K
RB = 80
NRC = N // RB
RC_PER_TILE = -(-NRC // NS)
ZR = 16

ROWS_B = 1000
NBLK = N // ROWS_B


_sc_mesh = plsc.VectorSubcoreMesh(core_axis_name="c", subcore_axis_name="s")


@functools.partial(
    pl.kernel,
    out_type=jax.ShapeDtypeStruct((NC * N, D), jnp.float32),
    mesh=_sc_mesh,
    scratch_types=[
        pltpu.VMEM((EPW,), jnp.int32),
        pltpu.VMEM((K,), jnp.int32),
        pltpu.VMEM((K,), jnp.int32),
        pltpu.VMEM((K, D), jnp.float32),
        pltpu.VMEM((K, D), jnp.float32),
        pltpu.VMEM((ZR, D), jnp.float32),
        pltpu.VMEM_SHARED((AGGR, D), jnp.float32),
        pltpu.SemaphoreType.DMA,
        pltpu.SemaphoreType.DMA,
        pltpu.SemaphoreType.DMA,
        pltpu.SemaphoreType.DMA,
    ],
)
def _sc_gather_segsum(msg_hbm, src_hbm, dst_hbm, out_hbm,
                      src_all, dst_c0, dst_c1, rows0, rows1, zbuf, agg_sh,
                      semd0, semd1, semg0, semg1):
    c = lax.axis_index("c")
    s = lax.axis_index("s")
    wid = s * NC + c
    base_w = pl.multiple_of(wid * EPW, EPW)

    pltpu.sync_copy(src_hbm.at[pl.ds(base_w, EPW)], src_all)

    def _issue(i, dst_c, rows, semd, semg):
        pltpu.async_copy(dst_hbm.at[pl.ds(base_w + pl.multiple_of(i * K, K), K)],
                         dst_c, semd)
        idx = src_all.at[pl.ds(pl.multiple_of(i * K, K), K)]
        pltpu.async_copy(msg_hbm.at[idx], rows, semg)

    def _drain_scatter(dst_c, rows, semd, semg):
        pltpu.make_async_copy(dst_hbm.at[pl.ds(0, K)], dst_c, semd).wait()
        pltpu.make_async_copy(msg_hbm.at[pl.ds(0, K)], rows, semg).wait()
        pltpu.sync_copy(rows, agg_sh.at[dst_c], add=True)

    _issue(0, dst_c0, rows0, semd0, semg0)
    _issue(1, dst_c1, rows1, semd1, semg1)

    zeros16 = jnp.zeros((16,), jnp.float32)

    def _zero_row(r, carry):
        for j in range(D // 16):
            zbuf[r, pl.ds(j * 16, 16)] = zeros16
        return carry

    lax.fori_loop(0, ZR, _zero_row, 0)

    def _zero_chunk(t, carry):
        j = s + t * NS
        pltpu.sync_copy(zbuf, agg_sh.at[pl.ds(pl.multiple_of(j * ZR, ZR), ZR)])
        return carry

    lax.fori_loop(0, (AGGR // ZR) // NS, _zero_chunk, 0)
    plsc.subcore_barrier()

    def _edge_pair(t, carry):
        _drain_scatter(dst_c0, rows0, semd0, semg0)
        _issue(2 * t + 2, dst_c0, rows0, semd0, semg0)
        _drain_scatter(dst_c1, rows1, semd1, semg1)
        _issue(2 * t + 3, dst_c1, rows1, semd1, semg1)
        return carry

    lax.fori_loop(0, NCHUNK // 2 - 1, _edge_pair, 0)
    _drain_scatter(dst_c0, rows0, semd0, semg0)
    _drain_scatter(dst_c1, rows1, semd1, semg1)
    plsc.subcore_barrier()

    def _write_chunk(t, carry):
        j = s + t * NS

        @pl.when(j < NRC)
        def _():
            base = pl.multiple_of(j * RB, RB)
            pltpu.sync_copy(agg_sh.at[pl.ds(base, RB)],
                            out_hbm.at[pl.ds(pl.multiple_of(c * N, RB) + base, RB)])

        return carry

    lax.fori_loop(0, RC_PER_TILE, _write_chunk, 0)


def _tc_a_body(x_ref, w_ref, im_ref, msg_ref):
    im = lax.dot_general(x_ref[...], w_ref[...], (((1,), (1,)), ((), ())),
                         preferred_element_type=jnp.float32)
    im_ref[...] = im
    msg_ref[...] = jnp.maximum(im, 0.0)


_tc_a = pl.pallas_call(
    _tc_a_body,
    grid=(NBLK,),
    in_specs=[
        pl.BlockSpec((ROWS_B, D), lambda i: (i, 0)),
        pl.BlockSpec((OUT, D), lambda i: (0, 0)),
    ],
    out_specs=[
        pl.BlockSpec((ROWS_B, OUT), lambda i: (i, 0)),
        pl.BlockSpec((ROWS_B, OUT), lambda i: (i, 0)),
    ],
    out_shape=[
        jax.ShapeDtypeStruct((N, OUT), jnp.float32),
        jax.ShapeDtypeStruct((N, OUT), jnp.float32),
    ],
)


def _tc_b_body(im_ref, a_ref, b_ref, w_ref, msg_ref):
    agg = a_ref[...] + b_ref[...]
    y = lax.dot_general(agg, w_ref[...], (((1,), (1,)), ((), ())),
                        preferred_element_type=jnp.float32)
    msg_ref[...] = jnp.maximum(im_ref[...] + y, 0.0)


_tc_b = pl.pallas_call(
    _tc_b_body,
    grid=(NBLK,),
    in_specs=[
        pl.BlockSpec((ROWS_B, OUT), lambda i: (i, 0)),
        pl.BlockSpec((ROWS_B, OUT), lambda i: (i, 0)),
        pl.BlockSpec((ROWS_B, OUT), lambda i: (i, 0)),
        pl.BlockSpec((OUT, OUT), lambda i: (0, 0)),
    ],
    out_specs=pl.BlockSpec((ROWS_B, OUT), lambda i: (i, 0)),
    out_shape=jax.ShapeDtypeStruct((N, OUT), jnp.float32),
)


def _tc_b_last_body(im_ref, a_ref, b_ref, w_ref, gid_ref, out_ref):
    i = pl.program_id(0)
    agg = a_ref[...] + b_ref[...]
    y = lax.dot_general(agg, w_ref[...], (((1,), (1,)), ((), ())),
                        preferred_element_type=jnp.float32)
    msg = jnp.maximum(im_ref[...] + y, 0.0)
    gid = gid_ref[0, 0, :]
    graphs = lax.broadcasted_iota(jnp.int32, (ROWS_B, G), 1)
    onehot = jnp.where(gid[:, None] == graphs, 1.0, 0.0).astype(jnp.float32)
    contrib = lax.dot_general(onehot, msg, (((0,), (0,)), ((), ())),
                              preferred_element_type=jnp.float32)

    @pl.when(i == 0)
    def _():
        out_ref[...] = jnp.zeros_like(out_ref)

    out_ref[...] += contrib


_tc_b_last = pl.pallas_call(
    _tc_b_last_body,
    grid=(NBLK,),
    in_specs=[
        pl.BlockSpec((ROWS_B, OUT), lambda i: (i, 0)),
        pl.BlockSpec((ROWS_B, OUT), lambda i: (i, 0)),
        pl.BlockSpec((ROWS_B, OUT), lambda i: (i, 0)),
        pl.BlockSpec((OUT, OUT), lambda i: (0, 0)),
        pl.BlockSpec((1, 1, ROWS_B), lambda i: (i, 0, 0)),
    ],
    out_specs=pl.BlockSpec((G, OUT), lambda i: (0, 0)),
    out_shape=jax.ShapeDtypeStruct((G, OUT), jnp.float32),
)


def kernel(node_feat, edge_index, graph_ids, W_n2l, W_rec):
    pad = jnp.arange(EPAD - E, dtype=jnp.int32)
    src = jnp.concatenate([edge_index[0], pad % N])
    dst = jnp.concatenate([edge_index[1], N + pad % (AGGR - N)])
    gid3 = graph_ids.reshape(NBLK, 1, ROWS_B)

    im, msg = _tc_a(node_feat, W_n2l)
    for step in range(STEPS):
        parts = _sc_gather_segsum(msg, src, dst)
        agg_a = parts[:N]
        agg_b = parts[N:]
        if step < STEPS - 1:
            msg = _tc_b(im, agg_a, agg_b, W_rec)
        else:
            out = _tc_b_last(im, agg_a, agg_b, W_rec, gid3)
    return out
